# baseline (speedup 1.0000x reference)
import jax
import jax.numpy as jnp
from jax.experimental import pallas as pl
from jax.experimental.pallas import tpu as pltpu

_NOISE = 0.1
_ROWS = 192 * 384 * 384 // 384
_W = 384
_B = 2048
_N = _ROWS // _B


def _sp_kernel(img_ref, noise_ref, out_ref, vmin_ref, vmax_ref, mm_ref):
    i = pl.program_id(0)

    @pl.when(i == 0)
    def _init():
        vmin_ref[...] = jnp.full((8, _W), jnp.inf, jnp.float32)
        vmax_ref[...] = jnp.full((8, _W), -jnp.inf, jnp.float32)

    @pl.when(i < _N)
    def _reduce():
        x = img_ref[...].reshape(_B // 8, 8, _W)
        vmin_ref[...] = jnp.minimum(vmin_ref[...], jnp.min(x, axis=0))
        vmax_ref[...] = jnp.maximum(vmax_ref[...], jnp.max(x, axis=0))

    @pl.when(i == _N)
    def _finalize():
        mm_ref[0] = jnp.min(vmin_ref[...])
        mm_ref[1] = jnp.max(vmax_ref[...])

    @pl.when(i >= _N)
    def _apply():
        x = img_ref[...]
        u = noise_ref[...]
        mn = mm_ref[0]
        mx = mm_ref[1]
        out = jnp.where(u < _NOISE, mn, x)
        out_ref[...] = jnp.where(u > 1.0 - _NOISE, mx, out)


def kernel(img, noise_u):
    x = img.reshape(_ROWS, _W)
    u = noise_u.reshape(_ROWS, _W)
    out = pl.pallas_call(
        _sp_kernel,
        grid=(2 * _N,),
        in_specs=[
            pl.BlockSpec((_B, _W), lambda i: (i % _N, 0)),
            pl.BlockSpec((_B, _W), lambda i: (jnp.where(i < _N, 0, i - _N), 0)),
        ],
        out_specs=pl.BlockSpec((_B, _W), lambda i: (jnp.where(i < _N, 0, i - _N), 0)),
        out_shape=jax.ShapeDtypeStruct((_ROWS, _W), jnp.float32),
        scratch_shapes=[
            pltpu.VMEM((8, _W), jnp.float32),
            pltpu.VMEM((8, _W), jnp.float32),
            pltpu.SMEM((2,), jnp.float32),
        ],
    )(x, u)
    return out.reshape(img.shape)


# speedup vs baseline: 1.1292x; 1.1292x over previous
import jax
import jax.numpy as jnp
from jax.experimental import pallas as pl
from jax.experimental.pallas import tpu as pltpu

_NOISE = 0.1
_ROWS = 192 * 384 * 384 // 384
_W = 384
_B = 4096
_N = _ROWS // _B
_K = 4


def _sp_kernel(img_ref, noise_ref, out_ref, vmin_ref, vmax_ref, park_ref, mm_ref,
               dma_sem):
    i = pl.program_id(0)

    @pl.when(i == 0)
    def _init():
        vmin_ref[...] = jnp.full((8, _W), jnp.inf, jnp.float32)
        vmax_ref[...] = jnp.full((8, _W), -jnp.inf, jnp.float32)

    @pl.when(i < _N)
    def _reduce():
        @pl.when(i >= _N - _K)
        def _park_start():
            pltpu.make_async_copy(
                img_ref,
                park_ref.at[i - (_N - _K)],
                dma_sem,
            ).start()

        for c in range(0, _B, 512):
            xr = img_ref[pl.ds(c, 512), :].reshape(64, 8, _W)
            vmin_ref[...] = jnp.minimum(vmin_ref[...], jnp.min(xr, axis=0))
            vmax_ref[...] = jnp.maximum(vmax_ref[...], jnp.max(xr, axis=0))

        @pl.when(i >= _N - _K)
        def _park_wait():
            pltpu.make_async_copy(
                img_ref,
                park_ref.at[i - (_N - _K)],
                dma_sem,
            ).wait()

    @pl.when(i == _N)
    def _finalize():
        mm_ref[0] = jnp.min(vmin_ref[...])
        mm_ref[1] = jnp.max(vmax_ref[...])

    @pl.when(i >= _N)
    def _apply():
        j = i - _N
        mn = mm_ref[0]
        mx = mm_ref[1]

        @pl.when(j < _N - _K)
        def _from_hbm():
            for c in range(0, _B, 512):
                x = img_ref[pl.ds(c, 512), :]
                uc = noise_ref[pl.ds(c, 512), :]
                out = jnp.where(uc < _NOISE, mn, x)
                out_ref[pl.ds(c, 512), :] = jnp.where(uc > 1.0 - _NOISE, mx, out)

        @pl.when(j >= _N - _K)
        def _from_park():
            for c in range(0, _B, 512):
                x = park_ref[j - (_N - _K), pl.ds(c, 512), :]
                uc = noise_ref[pl.ds(c, 512), :]
                out = jnp.where(uc < _NOISE, mn, x)
                out_ref[pl.ds(c, 512), :] = jnp.where(uc > 1.0 - _NOISE, mx, out)


def kernel(img, noise_u):
    x = img.reshape(_ROWS, _W)
    u = noise_u.reshape(_ROWS, _W)
    out = pl.pallas_call(
        _sp_kernel,
        grid=(2 * _N,),
        in_specs=[
            pl.BlockSpec(
                (_B, _W),
                lambda i: (jnp.where(i < _N, i, jnp.minimum(i - _N, _N - _K - 1)), 0),
            ),
            pl.BlockSpec((_B, _W), lambda i: (jnp.where(i < _N, 0, i - _N), 0)),
        ],
        out_specs=pl.BlockSpec((_B, _W), lambda i: (jnp.where(i < _N, 0, i - _N), 0)),
        out_shape=jax.ShapeDtypeStruct((_ROWS, _W), jnp.float32),
        compiler_params=pltpu.CompilerParams(vmem_limit_bytes=67000000),
        scratch_shapes=[
            pltpu.VMEM((8, _W), jnp.float32),
            pltpu.VMEM((8, _W), jnp.float32),
            pltpu.VMEM((_K, _B, _W), jnp.float32),
            pltpu.SMEM((2,), jnp.float32),
            pltpu.SemaphoreType.DMA,
        ],
    )(x, u)
    return out.reshape(img.shape)


# speedup vs baseline: 1.1377x; 1.0075x over previous
import jax
import jax.numpy as jnp
from jax.experimental import pallas as pl
from jax.experimental.pallas import tpu as pltpu

_NOISE = 0.1
_ROWS = 192 * 384 * 384 // 384
_W = 384
_B = 3072
_N = _ROWS // _B
_K = 8


def _sp_kernel(img_ref, noise_ref, out_ref, vmin_ref, vmax_ref, park_ref, mm_ref,
               dma_sem):
    i = pl.program_id(0)

    @pl.when(i == 0)
    def _init():
        vmin_ref[...] = jnp.full((8, _W), jnp.inf, jnp.float32)
        vmax_ref[...] = jnp.full((8, _W), -jnp.inf, jnp.float32)

    @pl.when(i < _N)
    def _reduce():
        @pl.when(i >= _N - _K)
        def _park_start():
            pltpu.make_async_copy(
                img_ref,
                park_ref.at[i - (_N - _K)],
                dma_sem,
            ).start()

        for c in range(0, _B, 512):
            xr = img_ref[pl.ds(c, 512), :].reshape(64, 8, _W)
            vmin_ref[...] = jnp.minimum(vmin_ref[...], jnp.min(xr, axis=0))
            vmax_ref[...] = jnp.maximum(vmax_ref[...], jnp.max(xr, axis=0))

        @pl.when(i >= _N - _K)
        def _park_wait():
            pltpu.make_async_copy(
                img_ref,
                park_ref.at[i - (_N - _K)],
                dma_sem,
            ).wait()

    @pl.when(i == _N)
    def _finalize():
        mm_ref[0] = jnp.min(vmin_ref[...])
        mm_ref[1] = jnp.max(vmax_ref[...])

    @pl.when(i >= _N)
    def _apply():
        j = i - _N
        mn = mm_ref[0]
        mx = mm_ref[1]

        @pl.when(j < _N - _K)
        def _from_hbm():
            for c in range(0, _B, 512):
                x = img_ref[pl.ds(c, 512), :]
                uc = noise_ref[pl.ds(c, 512), :]
                out = jnp.where(uc < _NOISE, mn, x)
                out_ref[pl.ds(c, 512), :] = jnp.where(uc > 1.0 - _NOISE, mx, out)

        @pl.when(j >= _N - _K)
        def _from_park():
            for c in range(0, _B, 512):
                x = park_ref[j - (_N - _K), pl.ds(c, 512), :]
                uc = noise_ref[pl.ds(c, 512), :]
                out = jnp.where(uc < _NOISE, mn, x)
                out_ref[pl.ds(c, 512), :] = jnp.where(uc > 1.0 - _NOISE, mx, out)


def kernel(img, noise_u):
    x = img.reshape(_ROWS, _W)
    u = noise_u.reshape(_ROWS, _W)
    out = pl.pallas_call(
        _sp_kernel,
        grid=(2 * _N,),
        in_specs=[
            pl.BlockSpec(
                (_B, _W),
                lambda i: (jnp.where(i < _N, i, jnp.minimum(i - _N, _N - _K - 1)), 0),
            ),
            pl.BlockSpec((_B, _W), lambda i: (jnp.where(i < _N, 0, i - _N), 0)),
        ],
        out_specs=pl.BlockSpec((_B, _W), lambda i: (jnp.where(i < _N, 0, i - _N), 0)),
        out_shape=jax.ShapeDtypeStruct((_ROWS, _W), jnp.float32),
        compiler_params=pltpu.CompilerParams(vmem_limit_bytes=67000000),
        scratch_shapes=[
            pltpu.VMEM((8, _W), jnp.float32),
            pltpu.VMEM((8, _W), jnp.float32),
            pltpu.VMEM((_K, _B, _W), jnp.float32),
            pltpu.SMEM((2,), jnp.float32),
            pltpu.SemaphoreType.DMA,
        ],
    )(x, u)
    return out.reshape(img.shape)
